# Initial kernel scaffold; baseline (speedup 1.0000x reference)
import jax
import jax.numpy as jnp
from jax.experimental import pallas as pl


def kernel(input_bitarrays, batch_size, adjacency_matrix_logits, invert_logits):
    raise NotImplementedError("write your pallas kernel here")


# speedup vs baseline: 1.5521x; 1.5521x over previous
import numpy as np
import jax
import jax.numpy as jnp
from jax.experimental import pallas as pl
from jax.experimental.pallas import tpu as pltpu

NUM_IN = 1024
NUM_OUT = 2048
WIDTH = 1024
BATCH = 8
OB = 128


def _threefry_keys():
    def tf(k1, k2, x0, x1):
        ks = [np.uint32(k1), np.uint32(k2)]
        ks.append(np.uint32(ks[0] ^ ks[1] ^ np.uint32(0x1BD11BDA)))
        rot = [(13, 15, 26, 6), (17, 29, 16, 24)]
        x = [np.uint32(x0), np.uint32(x1)]
        x[0] += ks[0]
        x[1] += ks[1]
        for i, (inj_a, inj_b) in enumerate([(1, 2), (2, 0), (0, 1), (1, 2), (2, 0)]):
            for r in rot[i % 2]:
                x[0] += x[1]
                x[1] = np.uint32((x[1] << np.uint32(r)) | (x[1] >> np.uint32(32 - r)))
                x[1] ^= x[0]
            x[0] += ks[inj_a]
            x[1] += ks[inj_b] + np.uint32(i + 1)
        return x[0], x[1]

    with np.errstate(over="ignore"):
        a0, b0 = tf(0, 1234, 0, 0)
        a1, b1 = tf(0, 1234, 0, 1)
    return (a0, a1), (b0, b1)


(_KS1_A, _KS2_A), (_KS1_B, _KS2_B) = _threefry_keys()
_TINY = np.float32(np.finfo(np.float32).tiny)
_EXP_ONE = np.uint32(0x3F800000)


def _bits(k1, k2, cnt_lo):
    u32 = jnp.uint32
    ks0 = u32(k1)
    ks1 = u32(k2)
    ks2 = u32(np.uint32(k1) ^ np.uint32(k2) ^ np.uint32(0x1BD11BDA))
    ks = (ks0, ks1, ks2)
    rot = ((13, 15, 26, 6), (17, 29, 16, 24))
    x0 = jnp.zeros_like(cnt_lo) + ks0
    x1 = cnt_lo + ks1
    for i, (inj_a, inj_b) in enumerate(((1, 2), (2, 0), (0, 1), (1, 2), (2, 0))):
        for r in rot[i % 2]:
            x0 = x0 + x1
            x1 = (x1 << u32(r)) | (x1 >> u32(32 - r))
            x1 = x0 ^ x1
        x0 = x0 + ks[inj_a]
        x1 = x1 + ks[inj_b] + u32(i + 1)
    return x0 ^ x1


def _uniform_from_bits(bits):
    fb = (bits >> jnp.uint32(9)) | jnp.uint32(_EXP_ONE)
    return jax.lax.bitcast_convert_type(fb, jnp.float32) - jnp.float32(1.0)


def _body(logits_ref, table_ref, p_ref, out_ref, idx0_ref, idx1_ref, mask_ref):
    b = pl.program_id(0)
    ob = pl.program_id(1)

    logits = logits_ref[...].reshape(2 * OB, NUM_IN)
    rows = jax.lax.broadcasted_iota(jnp.int32, (2 * OB, NUM_IN), 0)
    cols = jax.lax.broadcasted_iota(jnp.int32, (2 * OB, NUM_IN), 1)
    r_glob = jnp.where(rows < OB, ob * OB + rows, NUM_OUT - OB + ob * OB + rows)
    cnt = ((b * (2 * NUM_OUT) + r_glob) * NUM_IN + cols).astype(jnp.uint32)

    f = _uniform_from_bits(_bits(_KS1_A, _KS1_B, cnt))
    u = jnp.maximum(_TINY, f * (jnp.float32(1.0) - _TINY) + _TINY)
    g = -jnp.log(-jnp.log(u)) + logits

    m = jnp.max(g, axis=1, keepdims=True)
    idxc = jnp.min(jnp.where(g == m, cols, NUM_IN), axis=1, keepdims=True)
    idxc = idxc.astype(jnp.int32)

    oh = (cols == idxc).astype(jnp.bfloat16)
    ohs = oh[:OB] + oh[OB:]
    s = jax.lax.dot_general(
        ohs, table_ref[...], (((1,), (0,)), ((), ())),
        preferred_element_type=jnp.float32)
    anded = (s == jnp.float32(2.0)).astype(jnp.int32)

    ocol = jax.lax.broadcasted_iota(jnp.int32, (OB, 1), 0) + ob * OB
    mf = _uniform_from_bits(_bits(_KS2_A, _KS2_B, ocol.astype(jnp.uint32)))
    minv = jnp.where(mf < p_ref[...], jnp.int32(-1), jnp.int32(0))

    out_ref[...] = (anded ^ minv).reshape(1, OB, WIDTH)
    idx0_ref[...] = idxc[:OB].reshape(1, OB, 1)
    idx1_ref[...] = idxc[OB:].reshape(1, OB, 1)
    mask_ref[...] = minv.reshape(1, OB, 1)


def _run(table_bf16, adjacency_matrix_logits, p_col):
    nb = NUM_OUT // OB
    grid = (BATCH, nb)
    out_shapes = (
        jax.ShapeDtypeStruct((BATCH, NUM_OUT, WIDTH), jnp.int32),
        jax.ShapeDtypeStruct((BATCH, NUM_OUT, 1), jnp.int32),
        jax.ShapeDtypeStruct((BATCH, NUM_OUT, 1), jnp.int32),
        jax.ShapeDtypeStruct((BATCH, NUM_OUT, 1), jnp.int32),
    )
    return pl.pallas_call(
        _body,
        grid=grid,
        in_specs=[
            pl.BlockSpec((2, OB, NUM_IN), lambda b, ob: (0, ob, 0)),
            pl.BlockSpec((NUM_IN, WIDTH), lambda b, ob: (0, 0)),
            pl.BlockSpec((OB, 1), lambda b, ob: (ob, 0)),
        ],
        out_specs=(
            pl.BlockSpec((1, OB, WIDTH), lambda b, ob: (b, ob, 0)),
            pl.BlockSpec((1, OB, 1), lambda b, ob: (b, ob, 0)),
            pl.BlockSpec((1, OB, 1), lambda b, ob: (b, ob, 0)),
            pl.BlockSpec((1, OB, 1), lambda b, ob: (b, ob, 0)),
        ),
        out_shape=out_shapes,
    )(adjacency_matrix_logits, table_bf16, p_col)


def kernel(input_bitarrays, batch_size, adjacency_matrix_logits, invert_logits):
    table_bf16 = input_bitarrays.astype(jnp.bfloat16)
    p_col = jax.nn.sigmoid(invert_logits).reshape(NUM_OUT, 1)
    func, idx0, idx1, maskc = _run(table_bf16, adjacency_matrix_logits, p_col)
    connection_indices = jnp.concatenate([idx0, idx1], axis=2)
    invert_mask = maskc[0].reshape(NUM_OUT) != 0
    return (func, connection_indices, invert_mask)
